# baseline (speedup 1.0000x reference)
import functools

import jax
import jax.numpy as jnp
from jax import lax
from jax.experimental import pallas as pl
from jax.experimental.pallas import tpu as pltpu
from jax.experimental.pallas import tpu_sc as plsc

N = 100000
E = 3200000
H = 8

NC = 2
NS = 16
NW = NC * NS

NT = 6272
NP = NS * NT
NP8 = NP * H
NT8 = NT * H

EG = 2048
G = 50
ET = G * EG
EP = NW * ET


def _f32(x):
    return jnp.asarray(x, jnp.float32)


def _tc_gru_xw_body(xT_ref, w0T_ref, wih_ref, whh_ref, bih_ref, bhh_ref,
                    out_ref):
    w0T = w0T_ref[:]
    giT = jnp.dot(wih_ref[:], w0T, preferred_element_type=jnp.float32) + bih_ref[:]
    ghT = jnp.dot(whh_ref[:], w0T, preferred_element_type=jnp.float32) + bhh_ref[:]
    rT = jax.nn.sigmoid(giT[0:8] + ghT[0:8])
    zT = jax.nn.sigmoid(giT[8:16] + ghT[8:16])
    nT = jnp.tanh(giT[16:24] + rT * ghT[16:24])
    wT = (1.0 - zT) * nT + zT * w0T
    out_ref[:] = jnp.dot(wT, xT_ref[:], preferred_element_type=jnp.float32)


def _tc_gru_xw(xT, w0T, wih, whh, bihc, bhhc):
    return pl.pallas_call(
        _tc_gru_xw_body,
        out_shape=jax.ShapeDtypeStruct((H, NP), jnp.float32),
    )(xT, w0T, wih, whh, bihc, bhhc)


def _tc_dinv_y_body(d0_ref, d1_ref, xwT_ref, y_ref, dinv_ref):
    dinv = lax.rsqrt(d0_ref[:] + d1_ref[:] + 1.0)
    dinv_ref[:] = dinv
    y_ref[:] = dinv * xwT_ref[:]


def _tc_dinv_y(d0, d1, xwT):
    return pl.pallas_call(
        _tc_dinv_y_body,
        out_shape=[jax.ShapeDtypeStruct((H, NP), jnp.float32),
                   jax.ShapeDtypeStruct((1, NP), jnp.float32)],
    )(d0, d1, xwT)


def _tc_head_body(s0_ref, s1_ref, dinv_ref, bg_ref, wl_ref, bl_ref, out_ref):
    h = dinv_ref[:] * (s0_ref[:] + s1_ref[:]) + bg_ref[:]
    h = jnp.maximum(h, 0.0)
    out_ref[:] = jnp.sum(h * wl_ref[:], axis=0, keepdims=True) + bl_ref[:]


def _tc_head(s0T, s1T, dinv, bg, wl, bl):
    return pl.pallas_call(
        _tc_head_body,
        out_shape=jax.ShapeDtypeStruct((1, NP), jnp.float32),
    )(s0T, s1T, dinv, bg, wl, bl)


def _sc_deg_body(col_hbm, ew_hbm, deg_out, deg_sp, colb, ewb, zb, sem):
    c = lax.axis_index("c")
    s = lax.axis_index("s")
    wid = c * NS + s

    def zloop(v, _):
        zb[pl.ds(v * 16, 16)] = jnp.zeros((16,), jnp.float32)
        return 0
    lax.fori_loop(0, NT // 16, zloop, 0)
    pltpu.sync_copy(zb, deg_sp.at[pl.ds(s * NT, NT)])
    plsc.subcore_barrier()

    def gloop(g, _):
        base = wid * ET + g * EG
        pltpu.sync_copy(col_hbm.at[pl.ds(base, EG)], colb)
        pltpu.sync_copy(ew_hbm.at[pl.ds(base, EG)], ewb)
        pltpu.async_copy(ewb, deg_sp.at[colb], sem, add=True).wait()
        return 0
    lax.fori_loop(0, G, gloop, 0)
    plsc.subcore_barrier()
    pltpu.sync_copy(deg_sp.at[pl.ds(s * NT, NT)],
                    deg_out.at[pl.ds(c * NP + s * NT, NT)])


def _sc_edges_body(row_hbm, col_hbm, ew_hbm, y_in,
                   s_out, s_sp,
                   ybuf, rowb0, colb0, ewb0, msgb0, rowb1, colb1, ewb1,
                   msgb1, gsem0, gsem1, ssem0, ssem1):
    c = lax.axis_index("c")
    s = lax.axis_index("s")
    wid = c * NS + s
    nb8 = s * NT8

    rowb = (rowb0, rowb1)
    colb = (colb0, colb1)
    ewb = (ewb0, ewb1)
    msgb = (msgb0, msgb1)
    gsem = (gsem0, gsem1)
    ssem = (ssem0, ssem1)

    NQ = NT8 // 4
    for q in range(4):
        pltpu.sync_copy(y_in.at[pl.ds(nb8 + q * NQ, NQ)], ybuf)

        def hloop(v, _):
            ybuf[pl.ds(v * 16, 16)] = ybuf[pl.ds(v * 16, 16)] * 0.5
            return 0
        lax.fori_loop(0, NQ // 16, hloop, 0)
        pltpu.sync_copy(ybuf, s_sp.at[pl.ds(nb8 + q * NQ, NQ)])
    plsc.subcore_barrier()

    def load_edges(g, p):
        base = wid * ET + g * EG
        pltpu.sync_copy(row_hbm.at[pl.ds(base, EG)], rowb[p])
        pltpu.sync_copy(col_hbm.at[pl.ds(base, EG)], colb[p])
        pltpu.sync_copy(ew_hbm.at[pl.ds(base, EG)], ewb[p])

    def fire_gathers(p):
        return [pltpu.async_copy(
            y_in.at[pl.ds(k * NP, NP)].at[rowb[p]],
            msgb[p].at[pl.ds(k * EG, EG)], gsem[p]) for k in range(H)]

    def fire_scatters(p):
        return [pltpu.async_copy(
            msgb[p].at[pl.ds(k * EG, EG)],
            s_sp.at[pl.ds(k * NP, NP)].at[colb[p]], ssem[p], add=True)
            for k in range(H)]

    def drain(ds_):
        for d in ds_:
            d.wait()

    def scale_k(p, k):
        def mloop(v, _):
            o = k * EG + v * 16
            msgb[p][pl.ds(o, 16)] = (msgb[p][pl.ds(o, 16)]
                                     * ewb[p][pl.ds(v * 16, 16)])
            return 0
        lax.fori_loop(0, EG // 16, mloop, 0)

    def gloop(g2, _):
        ga = 2 * g2
        gb = ga + 1
        load_edges(ga, 0)
        gda = fire_gathers(0)
        load_edges(gb, 1)
        gdb = fire_gathers(1)
        for k in range(H):
            gda[k].wait()
            scale_k(0, k)
        sda = fire_scatters(0)
        for k in range(H):
            gdb[k].wait()
            scale_k(1, k)
        sdb = fire_scatters(1)
        drain(sda)
        drain(sdb)
        return 0
    lax.fori_loop(0, G // 2, gloop, 0)
    plsc.subcore_barrier()

    pltpu.sync_copy(s_sp.at[pl.ds(nb8, NT8)],
                    s_out.at[pl.ds(c * NP8 + nb8, NT8)])


@functools.lru_cache(maxsize=1)
def _build_sc_kernels():
    mesh = plsc.VectorSubcoreMesh(core_axis_name="c", subcore_axis_name="s",
                                  num_cores=NC, num_subcores=NS)
    sc_deg = pl.kernel(
        _sc_deg_body,
        out_type=jax.ShapeDtypeStruct((NC * NP,), jnp.float32),
        mesh=mesh,
        scratch_types=[
            pltpu.VMEM_SHARED((NP,), jnp.float32),
            pltpu.VMEM((EG,), jnp.int32),
            pltpu.VMEM((EG,), jnp.float32),
            pltpu.VMEM((NT,), jnp.float32),
            pltpu.SemaphoreType.DMA,
        ],
    )
    sc_edges = pl.kernel(
        _sc_edges_body,
        out_type=jax.ShapeDtypeStruct((NC * NP8,), jnp.float32),
        mesh=mesh,
        scratch_types=[
            pltpu.VMEM_SHARED((NP8,), jnp.float32),
            pltpu.VMEM((NT8 // 4,), jnp.float32),
            pltpu.VMEM((EG,), jnp.int32),
            pltpu.VMEM((EG,), jnp.int32),
            pltpu.VMEM((EG,), jnp.float32),
            pltpu.VMEM((H * EG,), jnp.float32),
            pltpu.VMEM((EG,), jnp.int32),
            pltpu.VMEM((EG,), jnp.int32),
            pltpu.VMEM((EG,), jnp.float32),
            pltpu.VMEM((H * EG,), jnp.float32),
            pltpu.SemaphoreType.DMA,
            pltpu.SemaphoreType.DMA,
            pltpu.SemaphoreType.DMA,
            pltpu.SemaphoreType.DMA,
        ],
    )
    return sc_deg, sc_edges


def kernel(x, edge_index, edge_weight, W0, W_ih, W_hh, b_ih, b_hh, b_gcn,
           W_lin, b_lin):
    x = _f32(x)
    ew = _f32(edge_weight)
    row = edge_index[0].astype(jnp.int32)
    col = edge_index[1].astype(jnp.int32)

    xT = jnp.pad(x, ((0, NP - N), (0, 0))).T
    pad = EP - E
    padn = (jnp.arange(pad, dtype=jnp.int32) * 131) % N
    rowp = jnp.concatenate([row, padn])
    colp = jnp.concatenate([col, padn])
    ewp = jnp.concatenate([ew, jnp.zeros((pad,), jnp.float32)])

    xwT = _tc_gru_xw(xT, _f32(W0).T, _f32(W_ih), _f32(W_hh),
                     _f32(b_ih).reshape(24, 1), _f32(b_hh).reshape(24, 1))

    sc_deg, sc_edges = _build_sc_kernels()
    degp = sc_deg(colp, ewp)

    yT, dinv = _tc_dinv_y(degp[:NP].reshape(1, NP), degp[NP:].reshape(1, NP),
                          xwT)

    ybig = jnp.concatenate([yT.reshape(NP8),
                            jnp.zeros((NP8,), jnp.float32)])
    sT = sc_edges(rowp, colp, ewp, ybig)

    out1 = _tc_head(sT[:NP8].reshape(H, NP), sT[NP8:].reshape(H, NP), dinv,
                    _f32(b_gcn).reshape(H, 1), _f32(W_lin).reshape(H, 1),
                    _f32(b_lin).reshape(1, 1))
    return out1.reshape(NP)[:N].reshape(N, 1)


# speedup vs baseline: 1.0721x; 1.0721x over previous
import functools

import jax
import jax.numpy as jnp
from jax import lax
from jax.experimental import pallas as pl
from jax.experimental.pallas import tpu as pltpu
from jax.experimental.pallas import tpu_sc as plsc

N = 100000
E = 3200000
H = 8

NC = 2
NS = 16
NW = NC * NS

NT = 6272
NP = NS * NT
NP8 = NP * H
NT8 = NT * H

EG = 2048
G = 50
ET = G * EG
EP = NW * ET


def _f32(x):
    return jnp.asarray(x, jnp.float32)


def _tc_gru_xw_body(xT_ref, w0T_ref, wih_ref, whh_ref, bih_ref, bhh_ref,
                    out_ref):
    w0T = w0T_ref[:]
    giT = jnp.dot(wih_ref[:], w0T, preferred_element_type=jnp.float32) + bih_ref[:]
    ghT = jnp.dot(whh_ref[:], w0T, preferred_element_type=jnp.float32) + bhh_ref[:]
    rT = jax.nn.sigmoid(giT[0:8] + ghT[0:8])
    zT = jax.nn.sigmoid(giT[8:16] + ghT[8:16])
    nT = jnp.tanh(giT[16:24] + rT * ghT[16:24])
    wT = (1.0 - zT) * nT + zT * w0T
    out_ref[:] = jnp.dot(wT, xT_ref[:], preferred_element_type=jnp.float32)


def _tc_gru_xw(xT, w0T, wih, whh, bihc, bhhc):
    return pl.pallas_call(
        _tc_gru_xw_body,
        out_shape=jax.ShapeDtypeStruct((H, NP), jnp.float32),
    )(xT, w0T, wih, whh, bihc, bhhc)


def _tc_dinv_y_body(d0_ref, d1_ref, xwT_ref, y_ref, dinv_ref):
    dinv = lax.rsqrt(d0_ref[:] + d1_ref[:] + 1.0)
    dinv_ref[:] = dinv
    y_ref[:] = dinv * xwT_ref[:]


def _tc_dinv_y(d0, d1, xwT):
    return pl.pallas_call(
        _tc_dinv_y_body,
        out_shape=[jax.ShapeDtypeStruct((H, NP), jnp.float32),
                   jax.ShapeDtypeStruct((1, NP), jnp.float32)],
    )(d0, d1, xwT)


def _tc_head_body(s0_ref, s1_ref, dinv_ref, bg_ref, wl_ref, bl_ref, out_ref):
    h = dinv_ref[:] * (s0_ref[:] + s1_ref[:]) + bg_ref[:]
    h = jnp.maximum(h, 0.0)
    out_ref[:] = jnp.sum(h * wl_ref[:], axis=0, keepdims=True) + bl_ref[:]


def _tc_head(s0T, s1T, dinv, bg, wl, bl):
    return pl.pallas_call(
        _tc_head_body,
        out_shape=jax.ShapeDtypeStruct((1, NP), jnp.float32),
    )(s0T, s1T, dinv, bg, wl, bl)


def _sc_deg_body(col_hbm, ew_hbm, deg_out, deg_sp, colb, ewb, colb2, ewb2,
                 zb, sem):
    c = lax.axis_index("c")
    s = lax.axis_index("s")
    wid = c * NS + s

    def zloop(v, _):
        zb[pl.ds(v * 16, 16)] = jnp.zeros((16,), jnp.float32)
        return 0
    lax.fori_loop(0, NT // 16, zloop, 0)
    pltpu.sync_copy(zb, deg_sp.at[pl.ds(s * NT, NT)])
    plsc.subcore_barrier()

    def gloop(g2, _):
        ba = wid * ET + (2 * g2) * EG
        bb = ba + EG
        pltpu.sync_copy(col_hbm.at[pl.ds(ba, EG)], colb)
        pltpu.sync_copy(ew_hbm.at[pl.ds(ba, EG)], ewb)
        da = pltpu.async_copy(ewb, deg_sp.at[colb], sem, add=True)
        pltpu.sync_copy(col_hbm.at[pl.ds(bb, EG)], colb2)
        pltpu.sync_copy(ew_hbm.at[pl.ds(bb, EG)], ewb2)
        db = pltpu.async_copy(ewb2, deg_sp.at[colb2], sem, add=True)
        da.wait()
        db.wait()
        return 0
    lax.fori_loop(0, G // 2, gloop, 0)
    plsc.subcore_barrier()
    pltpu.sync_copy(deg_sp.at[pl.ds(s * NT, NT)],
                    deg_out.at[pl.ds(c * NP + s * NT, NT)])


def _sc_edges_body(row_hbm, col_hbm, ew_hbm, y_in,
                   s_out, s_sp,
                   ybuf, rowb0, colb0, ewb0, msgb0, rowb1, colb1, ewb1,
                   msgb1, gsem0, gsem1, ssem0, ssem1):
    c = lax.axis_index("c")
    s = lax.axis_index("s")
    wid = c * NS + s
    nb8 = s * NT8

    rowb = (rowb0, rowb1)
    colb = (colb0, colb1)
    ewb = (ewb0, ewb1)
    msgb = (msgb0, msgb1)
    gsem = (gsem0, gsem1)
    ssem = (ssem0, ssem1)

    NQ = NT8 // 4
    for q in range(4):
        pltpu.sync_copy(y_in.at[pl.ds(nb8 + q * NQ, NQ)], ybuf)

        def hloop(v, _):
            ybuf[pl.ds(v * 16, 16)] = ybuf[pl.ds(v * 16, 16)] * 0.5
            return 0
        lax.fori_loop(0, NQ // 16, hloop, 0)
        pltpu.sync_copy(ybuf, s_sp.at[pl.ds(nb8 + q * NQ, NQ)])
    plsc.subcore_barrier()

    def load_edges(g, p):
        base = wid * ET + g * EG
        pltpu.sync_copy(row_hbm.at[pl.ds(base, EG)], rowb[p])
        pltpu.sync_copy(col_hbm.at[pl.ds(base, EG)], colb[p])
        pltpu.sync_copy(ew_hbm.at[pl.ds(base, EG)], ewb[p])

    def fire_gathers(p):
        return [pltpu.async_copy(
            y_in.at[pl.ds(k * NP, NP)].at[rowb[p]],
            msgb[p].at[pl.ds(k * EG, EG)], gsem[p]) for k in range(H)]

    def scale(p):
        def mloop(v, _):
            ev = ewb[p][pl.ds(v * 16, 16)]
            for k in range(H):
                o = k * EG + v * 16
                msgb[p][pl.ds(o, 16)] = msgb[p][pl.ds(o, 16)] * ev
            return 0
        lax.fori_loop(0, EG // 16, mloop, 0)

    def fire_scatters(p):
        return [pltpu.async_copy(
            msgb[p].at[pl.ds(k * EG, EG)],
            s_sp.at[pl.ds(k * NP, NP)].at[colb[p]], ssem[p], add=True)
            for k in range(H)]

    def drain(ds_):
        for d in ds_:
            d.wait()

    def gloop(g2, _):
        ga = 2 * g2
        gb = ga + 1
        load_edges(ga, 0)
        gda = fire_gathers(0)
        load_edges(gb, 1)
        gdb = fire_gathers(1)
        drain(gda)
        scale(0)
        sda = fire_scatters(0)
        drain(gdb)
        scale(1)
        sdb = fire_scatters(1)
        drain(sda)
        drain(sdb)
        return 0
    lax.fori_loop(0, G // 2, gloop, 0)
    plsc.subcore_barrier()

    pltpu.sync_copy(s_sp.at[pl.ds(nb8, NT8)],
                    s_out.at[pl.ds(c * NP8 + nb8, NT8)])


@functools.lru_cache(maxsize=1)
def _build_sc_kernels():
    mesh = plsc.VectorSubcoreMesh(core_axis_name="c", subcore_axis_name="s",
                                  num_cores=NC, num_subcores=NS)
    sc_deg = pl.kernel(
        _sc_deg_body,
        out_type=jax.ShapeDtypeStruct((NC * NP,), jnp.float32),
        mesh=mesh,
        scratch_types=[
            pltpu.VMEM_SHARED((NP,), jnp.float32),
            pltpu.VMEM((EG,), jnp.int32),
            pltpu.VMEM((EG,), jnp.float32),
            pltpu.VMEM((EG,), jnp.int32),
            pltpu.VMEM((EG,), jnp.float32),
            pltpu.VMEM((NT,), jnp.float32),
            pltpu.SemaphoreType.DMA,
        ],
    )
    sc_edges = pl.kernel(
        _sc_edges_body,
        out_type=jax.ShapeDtypeStruct((NC * NP8,), jnp.float32),
        mesh=mesh,
        scratch_types=[
            pltpu.VMEM_SHARED((NP8,), jnp.float32),
            pltpu.VMEM((NT8 // 4,), jnp.float32),
            pltpu.VMEM((EG,), jnp.int32),
            pltpu.VMEM((EG,), jnp.int32),
            pltpu.VMEM((EG,), jnp.float32),
            pltpu.VMEM((H * EG,), jnp.float32),
            pltpu.VMEM((EG,), jnp.int32),
            pltpu.VMEM((EG,), jnp.int32),
            pltpu.VMEM((EG,), jnp.float32),
            pltpu.VMEM((H * EG,), jnp.float32),
            pltpu.SemaphoreType.DMA,
            pltpu.SemaphoreType.DMA,
            pltpu.SemaphoreType.DMA,
            pltpu.SemaphoreType.DMA,
        ],
    )
    return sc_deg, sc_edges


def kernel(x, edge_index, edge_weight, W0, W_ih, W_hh, b_ih, b_hh, b_gcn,
           W_lin, b_lin):
    x = _f32(x)
    ew = _f32(edge_weight)
    row = edge_index[0].astype(jnp.int32)
    col = edge_index[1].astype(jnp.int32)

    xT = jnp.pad(x, ((0, NP - N), (0, 0))).T
    pad = EP - E
    padn = (jnp.arange(pad, dtype=jnp.int32) * 131) % N
    rowp = jnp.concatenate([row, padn])
    colp = jnp.concatenate([col, padn])
    ewp = jnp.concatenate([ew, jnp.zeros((pad,), jnp.float32)])

    xwT = _tc_gru_xw(xT, _f32(W0).T, _f32(W_ih), _f32(W_hh),
                     _f32(b_ih).reshape(24, 1), _f32(b_hh).reshape(24, 1))

    sc_deg, sc_edges = _build_sc_kernels()
    degp = sc_deg(colp, ewp)

    yT, dinv = _tc_dinv_y(degp[:NP].reshape(1, NP), degp[NP:].reshape(1, NP),
                          xwT)

    ybig = jnp.concatenate([yT.reshape(NP8),
                            jnp.zeros((NP8,), jnp.float32)])
    sT = sc_edges(rowp, colp, ewp, ybig)

    out1 = _tc_head(sT[:NP8].reshape(H, NP), sT[NP8:].reshape(H, NP), dinv,
                    _f32(b_gcn).reshape(H, 1), _f32(W_lin).reshape(H, 1),
                    _f32(b_lin).reshape(1, 1))
    return out1.reshape(NP)[:N].reshape(N, 1)
